# Initial kernel scaffold; baseline (speedup 1.0000x reference)
import jax
import jax.numpy as jnp
from jax.experimental import pallas as pl


def kernel(subtokens_embed, node_ids, W_ih_f, W_hh_f, b_ih_f, b_hh_f, W_ih_b, W_hh_b, b_ih_b, b_hh_b):
    raise NotImplementedError("write your pallas kernel here")


# speedup vs baseline: 6.1536x; 6.1536x over previous
import jax
import jax.numpy as jnp
from jax.experimental import pallas as pl

L = 8
D = 128
H = 32
PAD_IDX = 1000


def _bigru_kernel(x_ref, ids_ref, wih_ref, whhf_ref, whhb_ref, bih_ref,
                  bhhf_ref, bhhb_ref, out_ref):
    x = x_ref[...]
    ids = ids_ref[...]
    wih = wih_ref[...]
    bih = bih_ref[...]
    whh_f = whhf_ref[...]
    whh_b = whhb_ref[...]
    bhh_f = bhhf_ref[...]
    bhh_b = bhhb_ref[...]

    xg = [
        jnp.dot(x[:, t * D:(t + 1) * D], wih,
                preferred_element_type=jnp.float32) + bih
        for t in range(L)
    ]

    not_pad = ids != PAD_IDX
    masks = []
    m = not_pad[:, 0:1]
    for t in range(L):
        if t > 0:
            m = jnp.logical_and(m, not_pad[:, t:t + 1])
        masks.append(m)

    def step(h, xg_dir, whh, bhh, mask):
        hg = jnp.dot(h, whh, preferred_element_type=jnp.float32) + bhh
        r = jax.nn.sigmoid(xg_dir[:, :H] + hg[:, :H])
        z = jax.nn.sigmoid(xg_dir[:, H:2 * H] + hg[:, H:2 * H])
        n = jnp.tanh(xg_dir[:, 2 * H:] + r * hg[:, 2 * H:])
        h_new = (1.0 - z) * n + z * h
        return jnp.where(mask, h_new, h)

    B = x.shape[0]
    h_f = jnp.zeros((B, H), jnp.float32)
    h_b = jnp.zeros((B, H), jnp.float32)
    for t in range(L):
        h_f = step(h_f, xg[t][:, :3 * H], whh_f, bhh_f, masks[t])
        s = L - 1 - t
        h_b = step(h_b, xg[s][:, 3 * H:], whh_b, bhh_b, masks[s])

    out_ref[...] = h_f + h_b


def kernel(subtokens_embed, node_ids, W_ih_f, W_hh_f, b_ih_f, b_hh_f,
           W_ih_b, W_hh_b, b_ih_b, b_hh_b):
    n = subtokens_embed.shape[0]
    x2 = subtokens_embed.reshape(n, L * D)

    wih = jnp.concatenate([W_ih_f.T, W_ih_b.T], axis=1)
    bih = jnp.concatenate([b_ih_f, b_ih_b])[None, :]
    whh_f = W_hh_f.T
    whh_b = W_hh_b.T
    bhh_f = b_hh_f[None, :]
    bhh_b = b_hh_b[None, :]

    B = 1000
    pad = (-n) % B
    if pad:
        x2 = jnp.pad(x2, ((0, pad), (0, 0)))
        node_ids = jnp.pad(node_ids, ((0, pad), (0, 0)),
                           constant_values=PAD_IDX)
    n_pad = n + pad
    grid = n_pad // B

    out = pl.pallas_call(
        _bigru_kernel,
        grid=(grid,),
        in_specs=[
            pl.BlockSpec((B, L * D), lambda i: (i, 0)),
            pl.BlockSpec((B, L), lambda i: (i, 0)),
            pl.BlockSpec((D, 6 * H), lambda i: (0, 0)),
            pl.BlockSpec((H, 3 * H), lambda i: (0, 0)),
            pl.BlockSpec((H, 3 * H), lambda i: (0, 0)),
            pl.BlockSpec((1, 6 * H), lambda i: (0, 0)),
            pl.BlockSpec((1, 3 * H), lambda i: (0, 0)),
            pl.BlockSpec((1, 3 * H), lambda i: (0, 0)),
        ],
        out_specs=pl.BlockSpec((B, H), lambda i: (i, 0)),
        out_shape=jax.ShapeDtypeStruct((n_pad, H), jnp.float32),
    )(x2, node_ids, wih, whh_f, whh_b, bih, bhh_f, bhh_b)
    return out[:n]


# speedup vs baseline: 15.5282x; 2.5234x over previous
import jax
import jax.numpy as jnp
from jax.experimental import pallas as pl

L = 8
D = 128
H = 32
PAD_IDX = 1000


def _sigmoid(a):
    return 0.5 * jnp.tanh(0.5 * a) + 0.5


def _bigru_kernel(x_ref, idsT_ref, wih_ref, whh_ref, bih_ref, bhh_ref,
                  out_ref):
    x = x_ref[...]
    wih = wih_ref[...]
    whh = whh_ref[...]
    bih = bih_ref[...]
    bhh = bhh_ref[...]

    xg = [
        jax.lax.dot_general(wih, x[:, t * D:(t + 1) * D],
                            (((1,), (1,)), ((), ())),
                            preferred_element_type=jnp.float32) + bih
        for t in range(L)
    ]

    not_pad = idsT_ref[0] != PAD_IDX
    masks = []
    m = not_pad[0:1, :]
    for t in range(L):
        if t > 0:
            m = jnp.logical_and(m, not_pad[t:t + 1, :])
        masks.append(m)

    def step(h, xg_dir, whh_dir, bhh_dir, mask):
        hg = jnp.dot(whh_dir, h, preferred_element_type=jnp.float32) + bhh_dir
        rz = _sigmoid(xg_dir[:2 * H] + hg[:2 * H])
        r = rz[:H]
        z = rz[H:]
        n = jnp.tanh(xg_dir[2 * H:] + r * hg[2 * H:])
        h_new = (1.0 - z) * n + z * h
        return jnp.where(mask, h_new, h)

    B = x.shape[0]
    h_f = jnp.zeros((H, B), jnp.float32)
    h_b = jnp.zeros((H, B), jnp.float32)
    for t in range(L):
        h_f = step(h_f, xg[t][:3 * H], whh[:3 * H], bhh[:3 * H], masks[t])
        s = L - 1 - t
        h_b = step(h_b, xg[s][3 * H:], whh[3 * H:], bhh[3 * H:], masks[s])

    out_ref[...] = jnp.swapaxes(h_f + h_b, 0, 1)


def kernel(subtokens_embed, node_ids, W_ih_f, W_hh_f, b_ih_f, b_hh_f,
           W_ih_b, W_hh_b, b_ih_b, b_hh_b):
    n = subtokens_embed.shape[0]
    x2 = subtokens_embed.reshape(n, L * D)

    wih = jnp.concatenate([W_ih_f, W_ih_b], axis=0)
    whh = jnp.concatenate([W_hh_f, W_hh_b], axis=0)
    bih = jnp.concatenate([b_ih_f, b_ih_b])[:, None]
    bhh = jnp.concatenate([b_hh_f, b_hh_b])[:, None]

    B = 1000
    pad = (-n) % B
    if pad:
        x2 = jnp.pad(x2, ((0, pad), (0, 0)))
        node_ids = jnp.pad(node_ids, ((0, pad), (0, 0)),
                           constant_values=PAD_IDX)
    n_pad = n + pad
    grid = n_pad // B
    ids3 = node_ids.reshape(grid, B, L).transpose(0, 2, 1)

    out = pl.pallas_call(
        _bigru_kernel,
        grid=(grid,),
        in_specs=[
            pl.BlockSpec((B, L * D), lambda i: (i, 0)),
            pl.BlockSpec((1, L, B), lambda i: (i, 0, 0)),
            pl.BlockSpec((6 * H, D), lambda i: (0, 0)),
            pl.BlockSpec((6 * H, H), lambda i: (0, 0)),
            pl.BlockSpec((6 * H, 1), lambda i: (0, 0)),
            pl.BlockSpec((6 * H, 1), lambda i: (0, 0)),
        ],
        out_specs=pl.BlockSpec((B, H), lambda i: (i, 0)),
        out_shape=jax.ShapeDtypeStruct((n_pad, H), jnp.float32),
    )(x2, ids3, wih, whh, bih, bhh)
    return out[:n]


# speedup vs baseline: 16.1207x; 1.0382x over previous
import jax
import jax.numpy as jnp
from jax.experimental import pallas as pl

L = 8
D = 128
H = 32
PAD_IDX = 1000


def _sigmoid(a):
    return 0.5 * jnp.tanh(0.5 * a) + 0.5


def _bigru_kernel(x_ref, idsT_ref, wih_ref, whh_ref, bih_ref, bhh_ref,
                  out_ref):
    x = x_ref[...]
    wih = wih_ref[...]
    whh = whh_ref[...]
    bih = bih_ref[...]
    bhh = bhh_ref[...]

    xg = [
        jax.lax.dot_general(wih, x[:, t * D:(t + 1) * D],
                            (((1,), (1,)), ((), ())),
                            preferred_element_type=jnp.float32) + bih
        for t in range(L)
    ]

    not_pad = idsT_ref[0] != PAD_IDX
    masks = []
    m = not_pad[0:1, :]
    for t in range(L):
        if t > 0:
            m = jnp.logical_and(m, not_pad[t:t + 1, :])
        masks.append(m)

    def step(h, xg_dir, whh_dir, bhh_dir, mask):
        hg = jnp.dot(whh_dir, h, preferred_element_type=jnp.float32) + bhh_dir
        rz = _sigmoid(xg_dir[:2 * H] + hg[:2 * H])
        r = rz[:H]
        z = rz[H:]
        n = jnp.tanh(xg_dir[2 * H:] + r * hg[2 * H:])
        h_new = (1.0 - z) * n + z * h
        return jnp.where(mask, h_new, h)

    B = x.shape[0]
    h_f = jnp.zeros((H, B), jnp.float32)
    h_b = jnp.zeros((H, B), jnp.float32)
    for t in range(L):
        h_f = step(h_f, xg[t][:3 * H], whh[:3 * H], bhh[:3 * H], masks[t])
        s = L - 1 - t
        h_b = step(h_b, xg[s][3 * H:], whh[3 * H:], bhh[3 * H:], masks[s])

    out_ref[...] = jnp.swapaxes(h_f + h_b, 0, 1)


def kernel(subtokens_embed, node_ids, W_ih_f, W_hh_f, b_ih_f, b_hh_f,
           W_ih_b, W_hh_b, b_ih_b, b_hh_b):
    n = subtokens_embed.shape[0]
    x2 = subtokens_embed.reshape(n, L * D)

    wih = jnp.concatenate([W_ih_f, W_ih_b], axis=0)
    whh = jnp.concatenate([W_hh_f, W_hh_b], axis=0)
    bih = jnp.concatenate([b_ih_f, b_ih_b])[:, None]
    bhh = jnp.concatenate([b_hh_f, b_hh_b])[:, None]

    B = 2000
    pad = (-n) % B
    if pad:
        x2 = jnp.pad(x2, ((0, pad), (0, 0)))
        node_ids = jnp.pad(node_ids, ((0, pad), (0, 0)),
                           constant_values=PAD_IDX)
    n_pad = n + pad
    grid = n_pad // B
    ids3 = node_ids.reshape(grid, B, L).transpose(0, 2, 1)

    out = pl.pallas_call(
        _bigru_kernel,
        grid=(grid,),
        in_specs=[
            pl.BlockSpec((B, L * D), lambda i: (i, 0)),
            pl.BlockSpec((1, L, B), lambda i: (i, 0, 0)),
            pl.BlockSpec((6 * H, D), lambda i: (0, 0)),
            pl.BlockSpec((6 * H, H), lambda i: (0, 0)),
            pl.BlockSpec((6 * H, 1), lambda i: (0, 0)),
            pl.BlockSpec((6 * H, 1), lambda i: (0, 0)),
        ],
        out_specs=pl.BlockSpec((B, H), lambda i: (i, 0)),
        out_shape=jax.ShapeDtypeStruct((n_pad, H), jnp.float32),
    )(x2, ids3, wih, whh, bih, bhh)
    return out[:n]


# speedup vs baseline: 23.1609x; 1.4367x over previous
import jax
import jax.numpy as jnp
from jax.experimental import pallas as pl
from jax.experimental.pallas import tpu as pltpu

L = 8
D = 128
H = 32
PAD_IDX = 1000


def _sigmoid(a):
    return 0.5 * jnp.tanh(0.5 * a) + 0.5


def _bigru_kernel(x_hbm, ids_ref, wih_ref, whh_ref, bih_ref, bhh_ref,
                  out_ref, xbuf, sems):
    B = out_ref.shape[0]
    i = pl.program_id(0)
    ngrid = pl.num_programs(0)

    def copies(slot, blk):
        return [
            pltpu.make_async_copy(
                x_hbm.at[pl.ds(blk * B, B), t, :],
                xbuf.at[slot, t],
                sems.at[slot, t],
            )
            for t in range(L)
        ]

    @pl.when(i == 0)
    def _prologue():
        for c in copies(0, i):
            c.start()

    @pl.when(i + 1 < ngrid)
    def _prefetch():
        for c in copies((i + 1) % 2, i + 1):
            c.start()

    for c in copies(i % 2, i):
        c.wait()

    slot = i % 2
    wih = wih_ref[...]
    whh = whh_ref[...]
    bih = bih_ref[...]
    bhh = bhh_ref[...]

    xg = [
        jax.lax.dot_general(wih, xbuf[slot, t],
                            (((1,), (1,)), ((), ())),
                            preferred_element_type=jnp.float32) + bih
        for t in range(L)
    ]

    not_pad = (ids_ref[...] != PAD_IDX).astype(jnp.float32)
    cum = [not_pad[:, 0:1]]
    for t in range(1, L):
        cum.append(cum[-1] * not_pad[:, t:t + 1])
    cum_t = jnp.swapaxes(jnp.concatenate(cum, axis=1), 0, 1)
    masks = [cum_t[t:t + 1, :] > 0.5 for t in range(L)]

    def step(h, xg_dir, whh_dir, bhh_dir, mask):
        hg = jnp.dot(whh_dir, h, preferred_element_type=jnp.float32) + bhh_dir
        rz = _sigmoid(xg_dir[:2 * H] + hg[:2 * H])
        r = rz[:H]
        z = rz[H:]
        n = jnp.tanh(xg_dir[2 * H:] + r * hg[2 * H:])
        h_new = (1.0 - z) * n + z * h
        return jnp.where(mask, h_new, h)

    h_f = jnp.zeros((H, B), jnp.float32)
    h_b = jnp.zeros((H, B), jnp.float32)
    for t in range(L):
        h_f = step(h_f, xg[t][:3 * H], whh[:3 * H], bhh[:3 * H], masks[t])
        s = L - 1 - t
        h_b = step(h_b, xg[s][3 * H:], whh[3 * H:], bhh[3 * H:], masks[s])

    out_ref[...] = jnp.swapaxes(h_f + h_b, 0, 1)


def kernel(subtokens_embed, node_ids, W_ih_f, W_hh_f, b_ih_f, b_hh_f,
           W_ih_b, W_hh_b, b_ih_b, b_hh_b):
    n = subtokens_embed.shape[0]

    wih = jnp.concatenate([W_ih_f, W_ih_b], axis=0)
    whh = jnp.concatenate([W_hh_f, W_hh_b], axis=0)
    bih = jnp.concatenate([b_ih_f, b_ih_b])[:, None]
    bhh = jnp.concatenate([b_hh_f, b_hh_b])[:, None]

    B = 2000
    pad = (-n) % B
    if pad:
        subtokens_embed = jnp.pad(subtokens_embed,
                                  ((0, pad), (0, 0), (0, 0)))
        node_ids = jnp.pad(node_ids, ((0, pad), (0, 0)),
                           constant_values=PAD_IDX)
    n_pad = n + pad
    grid = n_pad // B

    out = pl.pallas_call(
        _bigru_kernel,
        grid=(grid,),
        in_specs=[
            pl.BlockSpec(memory_space=pl.ANY),
            pl.BlockSpec((B, L), lambda i: (i, 0)),
            pl.BlockSpec((6 * H, D), lambda i: (0, 0)),
            pl.BlockSpec((6 * H, H), lambda i: (0, 0)),
            pl.BlockSpec((6 * H, 1), lambda i: (0, 0)),
            pl.BlockSpec((6 * H, 1), lambda i: (0, 0)),
        ],
        out_specs=pl.BlockSpec((B, H), lambda i: (i, 0)),
        out_shape=jax.ShapeDtypeStruct((n_pad, H), jnp.float32),
        scratch_shapes=[
            pltpu.VMEM((2, L, B, D), jnp.float32),
            pltpu.SemaphoreType.DMA((2, L)),
        ],
    )(subtokens_embed, node_ids, wih, whh, bih, bhh)
    if pad:
        out = out[:n]
    return out


# speedup vs baseline: 34.3898x; 1.4848x over previous
import jax
import jax.numpy as jnp
from jax.experimental import pallas as pl
from jax.experimental.pallas import tpu as pltpu

L = 8
D = 128
H = 32
PAD_IDX = 1000
W_SUB = 256
GROUP = 4


def _bigru_kernel(x_hbm, ids_ref, wih_ref, whh_ref, bcomb_ref, bhh2_ref,
                  out_ref, xbuf, sems):
    B = out_ref.shape[0]
    i = pl.program_id(0)
    ngrid = pl.num_programs(0)

    def copies(slot, blk):
        return [
            pltpu.make_async_copy(
                x_hbm.at[pl.ds(blk * B, B), t, :],
                xbuf.at[slot, t],
                sems.at[slot, t],
            )
            for t in range(L)
        ]

    @pl.when(i == 0)
    def _prologue():
        for c in copies(0, i):
            c.start()

    @pl.when(i + 1 < ngrid)
    def _prefetch():
        for c in copies((i + 1) % 2, i + 1):
            c.start()

    for c in copies(i % 2, i):
        c.wait()

    slot = i % 2
    wih = wih_ref[...]
    whh = whh_ref[...]
    bcomb = bcomb_ref[...]
    bhh2 = bhh2_ref[...]

    not_pad = ids_ref[0] != PAD_IDX
    masks = [not_pad[0:1, :]]
    for t in range(1, L):
        masks.append(jnp.logical_and(masks[-1], not_pad[t:t + 1, :]))

    def step(h, xg, whh_dir, bcomb_dir, bhh2_dir, mask):
        hg = jnp.dot(whh_dir, h.astype(jnp.bfloat16),
                     preferred_element_type=jnp.float32)
        g = xg + bcomb_dir
        rz = jnp.tanh(g[:2 * H] + hg[:2 * H])
        hgn2 = 0.5 * hg[2 * H:] + bhh2_dir
        s = g[2 * H:] + hgn2
        n = jnp.tanh(s + rz[:H] * hgn2)
        d = h - n
        e = 0.5 * d + n
        h_new = e + (0.5 * rz[H:]) * d
        return jnp.where(mask, h_new, h)

    def proj(wih_dir, t, j0, w):
        xt = xbuf[slot, t, pl.ds(j0, w), :].astype(jnp.bfloat16)
        return jax.lax.dot_general(wih_dir, xt, (((1,), (1,)), ((), ())),
                                   preferred_element_type=jnp.float32)

    for g0 in range(0, B, W_SUB * GROUP):
        tiles = [(j0, min(W_SUB, B - j0))
                 for j0 in range(g0, min(g0 + W_SUB * GROUP, B), W_SUB)]
        h_f = {j0: jnp.zeros((H, w), jnp.float32) for j0, w in tiles}
        h_b = {j0: jnp.zeros((H, w), jnp.float32) for j0, w in tiles}
        for t in range(L):
            s = L - 1 - t
            for j0, w in tiles:
                h_f[j0] = step(h_f[j0], proj(wih[:3 * H], t, j0, w),
                               whh[:3 * H], bcomb[:3 * H], bhh2[:H],
                               masks[t][:, j0:j0 + w])
                h_b[j0] = step(h_b[j0], proj(wih[3 * H:], s, j0, w),
                               whh[3 * H:], bcomb[3 * H:], bhh2[H:],
                               masks[s][:, j0:j0 + w])
        for j0, w in tiles:
            out_ref[j0:j0 + w, :] = jnp.swapaxes(h_f[j0] + h_b[j0], 0, 1)


def kernel(subtokens_embed, node_ids, W_ih_f, W_hh_f, b_ih_f, b_hh_f,
           W_ih_b, W_hh_b, b_ih_b, b_hh_b):
    n = subtokens_embed.shape[0]

    gate_scale = jnp.concatenate([jnp.full((2 * H,), 0.5),
                                  jnp.ones((H,))])[:, None]
    wih = jnp.concatenate([W_ih_f * gate_scale,
                           W_ih_b * gate_scale], axis=0).astype(jnp.bfloat16)
    whh = jnp.concatenate([W_hh_f * gate_scale,
                           W_hh_b * gate_scale],
                          axis=0).astype(jnp.bfloat16)

    def bc(b_ih, b_hh):
        return jnp.concatenate([0.5 * (b_ih[:2 * H] + b_hh[:2 * H]),
                                b_ih[2 * H:]])
    bcomb = jnp.concatenate([bc(b_ih_f, b_hh_f),
                             bc(b_ih_b, b_hh_b)])[:, None]
    bhh2 = jnp.concatenate([0.5 * b_hh_f[2 * H:],
                            0.5 * b_hh_b[2 * H:]])[:, None]

    B = 2000
    pad = (-n) % B
    if pad:
        subtokens_embed = jnp.pad(subtokens_embed,
                                  ((0, pad), (0, 0), (0, 0)))
        node_ids = jnp.pad(node_ids, ((0, pad), (0, 0)),
                           constant_values=PAD_IDX)
    n_pad = n + pad
    grid = n_pad // B
    ids3 = node_ids.reshape(grid, B, L).transpose(0, 2, 1)

    out = pl.pallas_call(
        _bigru_kernel,
        grid=(grid,),
        in_specs=[
            pl.BlockSpec(memory_space=pl.ANY),
            pl.BlockSpec((1, L, B), lambda i: (i, 0, 0)),
            pl.BlockSpec((6 * H, D), lambda i: (0, 0)),
            pl.BlockSpec((6 * H, H), lambda i: (0, 0)),
            pl.BlockSpec((6 * H, 1), lambda i: (0, 0)),
            pl.BlockSpec((2 * H, 1), lambda i: (0, 0)),
        ],
        out_specs=pl.BlockSpec((B, H), lambda i: (i, 0)),
        out_shape=jax.ShapeDtypeStruct((n_pad, H), jnp.float32),
        scratch_shapes=[
            pltpu.VMEM((2, L, B, D), jnp.float32),
            pltpu.SemaphoreType.DMA((2, L)),
        ],
    )(subtokens_embed, ids3, wih, whh, bcomb, bhh2)
    if pad:
        out = out[:n]
    return out
